# baseline (speedup 1.0000x reference)
import functools

import jax
import jax.numpy as jnp
from jax import lax
from jax.experimental import pallas as pl
from jax.experimental.pallas import tpu as pltpu
from jax.experimental.pallas import tpu_sc as plsc

_EPS = 1e-4
_N_ITER = 3
_PI = 3.14159265358979

_NC = 2
_NS = 16
_L = 16
_NW = _NC * _NS


def _acos(x):
    ax = jnp.abs(x)
    p = jnp.float32(-0.0012624911)
    p = p * ax + jnp.float32(0.0066700901)
    p = p * ax - jnp.float32(0.0170881256)
    p = p * ax + jnp.float32(0.0308918810)
    p = p * ax - jnp.float32(0.0501743046)
    p = p * ax + jnp.float32(0.0889789874)
    p = p * ax - jnp.float32(0.2145988016)
    p = p * ax + jnp.float32(1.5707963050)
    r = jnp.sqrt(jnp.maximum(1.0 - ax, 0.0)) * p
    return jnp.where(x < 0, jnp.float32(_PI) - r, r)


@functools.lru_cache(maxsize=None)
def _make_pipeline(n, C, d):
    rpw = n // _NW
    chunk = min(32, rpw)
    nchunk = rpw // chunk
    mesh = plsc.VectorSubcoreMesh(core_axis_name="c", subcore_axis_name="s")

    def _labels_body(logits_ref, labels_ref, counts_ref):
        lg = logits_ref[...]
        m = jnp.max(lg, axis=1, keepdims=True)
        e = jnp.exp(lg - m)
        p = e / jnp.sum(e, axis=1, keepdims=True)
        pm = jnp.max(p, axis=1, keepdims=True)
        ii = lax.broadcasted_iota(jnp.int32, (n, C), 1)
        lab = jnp.min(jnp.where(p >= pm, ii, jnp.int32(1 << 30)), axis=1,
                      keepdims=True)
        labels_ref[...] = lab
        counts_ref[...] = jnp.sum(jnp.where(ii == lab, 1.0, 0.0), axis=0,
                                  keepdims=True)

    labels_call = pl.pallas_call(
        _labels_body,
        out_shape=[jax.ShapeDtypeStruct((n, 1), jnp.int32),
                   jax.ShapeDtypeStruct((1, C), jnp.float32)],
    )

    def _zero_acc(acc):
        z = jnp.zeros((_L,), jnp.float32)

        def zbody(i):
            for j in range(d // _L):
                acc[i, pl.ds(j * _L, _L)] = z

        plsc.parallel_loop(0, C, unroll=2)(zbody)

    @functools.partial(
        pl.kernel,
        out_type=jax.ShapeDtypeStruct((_NW, C, d), jnp.float32),
        mesh=mesh,
        compiler_params=pltpu.CompilerParams(needs_layout_passes=False),
        scratch_types=[pltpu.VMEM((rpw,), jnp.int32),
                       pltpu.VMEM((2, chunk, d), jnp.float32),
                       pltpu.VMEM((C, d), jnp.float32),
                       pltpu.SemaphoreType.DMA,
                       pltpu.SemaphoreType.DMA],
    )
    def classsum_call(x_hbm, lab_hbm, out_hbm, labv, xbuf, acc, xs0, xs1):
        core = lax.axis_index("c")
        sub = lax.axis_index("s")
        wid = sub * _NC + core
        base = wid * rpw
        lane = lax.broadcasted_iota(jnp.int32, (_L,), 0)
        xsems = (xs0, xs1)
        pltpu.sync_copy(lab_hbm.at[pl.ds(base, rpw)], labv)

        def _start(ch):
            b = ch % 2
            return pltpu.async_copy(
                x_hbm.at[pl.ds(base + ch * chunk, chunk)], xbuf.at[b],
                xsems[b])

        cps = {0: _start(0)}
        _zero_acc(acc)
        for ch in range(nchunk):
            b = ch % 2
            if ch + 1 < nchunk:
                cps[ch + 1] = _start(ch + 1)
            cps.pop(ch).wait()

            def row_body(r):
                lv = plsc.load_gather(
                    labv, [jnp.full((_L,), ch * chunk, jnp.int32) + r])
                for j in range(d // _L):
                    plsc.addupdate_scatter(acc, [lv, lane + (j * _L)],
                                           xbuf[b, r, pl.ds(j * _L, _L)])

            plsc.parallel_loop(0, chunk, unroll=2)(row_body)
        pltpu.sync_copy(acc, out_hbm.at[wid])

    def _mu0_body(sums_ref, counts_ref, mu_ref):
        s = jnp.sum(sums_ref[...], axis=0)
        cnt = jnp.maximum(counts_ref[...], 1.0)
        mean = s / cnt
        nrm = jnp.sqrt(jnp.sum(mean * mean, axis=1, keepdims=True))
        mu_ref[...] = mean / jnp.maximum(nrm, 1e-12)

    mu0_call = pl.pallas_call(
        _mu0_body,
        out_shape=jax.ShapeDtypeStruct((C, d), jnp.float32),
    )

    def _rsqrt16(x):
        i = plsc.bitcast(x, jnp.int32)
        y = plsc.bitcast(jnp.int32(0x5F3759DF) - (i >> 1), jnp.float32)
        for _ in range(3):
            y = y * (1.5 - 0.5 * x * y * y)
        return y

    def _sqrt16(x):
        xs = jnp.maximum(x, 1e-24)
        return xs * _rsqrt16(xs)

    def _acos16(x):
        ax = jnp.abs(x)
        p = jnp.float32(-0.0012624911)
        p = p * ax + jnp.float32(0.0066700901)
        p = p * ax - jnp.float32(0.0170881256)
        p = p * ax + jnp.float32(0.0308918810)
        p = p * ax - jnp.float32(0.0501743046)
        p = p * ax + jnp.float32(0.0889789874)
        p = p * ax - jnp.float32(0.2145988016)
        p = p * ax + jnp.float32(1.5707963050)
        r = _sqrt16(jnp.maximum(1.0 - ax, 0.0)) * p
        return jnp.where(x < 0, jnp.float32(_PI) - r, r)

    @functools.partial(
        pl.kernel,
        out_type=[jax.ShapeDtypeStruct((_NW, C, d), jnp.float32),
                  jax.ShapeDtypeStruct((n,), jnp.float32)],
        mesh=mesh,
        compiler_params=pltpu.CompilerParams(needs_layout_passes=False),
        scratch_types=[pltpu.VMEM((nchunk, chunk), jnp.int32),
                       pltpu.VMEM((2, chunk, d), jnp.float32),
                       pltpu.VMEM((2, chunk, d), jnp.float32),
                       pltpu.VMEM((chunk,), jnp.float32),
                       pltpu.VMEM((chunk,), jnp.float32),
                       pltpu.VMEM((C, d), jnp.float32),
                       pltpu.SemaphoreType.DMA,
                       pltpu.SemaphoreType.DMA,
                       pltpu.SemaphoreType.DMA,
                       pltpu.SemaphoreType.DMA],
    )
    def karcher_call(x_hbm, lab_hbm, mu_hbm, vec_hbm, sc_hbm, idx_v, xbuf,
                     gbuf, sbuf, scbuf, acc, xs0, xs1, gs0, gs1):
        core = lax.axis_index("c")
        sub = lax.axis_index("s")
        wid = sub * _NC + core
        base = wid * rpw
        lane = lax.broadcasted_iota(jnp.int32, (_L,), 0)
        xsems = (xs0, xs1)
        gsems = (gs0, gs1)
        for ch in range(nchunk):
            pltpu.sync_copy(lab_hbm.at[pl.ds(base + ch * chunk, chunk)],
                            idx_v.at[ch])

        def _start(ch):
            b = ch % 2
            cx = pltpu.async_copy(
                x_hbm.at[pl.ds(base + ch * chunk, chunk)], xbuf.at[b],
                xsems[b])
            cg = pltpu.async_copy(mu_hbm.at[idx_v.at[ch]], gbuf.at[b],
                                  gsems[b])
            return cx, cg

        cps = {0: _start(0)}
        _zero_acc(acc)
        for ch in range(nchunk):
            b = ch % 2
            if ch + 1 < nchunk:
                cps[ch + 1] = _start(ch + 1)
            cx, cg = cps.pop(ch)
            cx.wait()
            cg.wait()
            for g in range(chunk // _L):
                def dot_body(r, vec):
                    row = g * _L + r
                    a = xbuf[b, row, pl.ds(0, _L)] * gbuf[b, row, pl.ds(0, _L)]
                    for j in range(1, d // _L):
                        a = a + (xbuf[b, row, pl.ds(j * _L, _L)] *
                                 gbuf[b, row, pl.ds(j * _L, _L)])
                    return jnp.where(lane == r, jnp.sum(a), vec)
                dvec = plsc.parallel_loop(
                    0, _L, carry=jnp.zeros((_L,), jnp.float32))(dot_body)
                cv = jnp.clip(dvec, -1.0 + _EPS, 1.0 - _EPS)
                theta = _acos16(cv)
                dn2 = jnp.maximum(1.0 - 2.0 * cv * dvec + cv * cv, 1e-8)
                sv = theta * _rsqrt16(dn2)
                sbuf[pl.ds(g * _L, _L)] = sv
                scbuf[pl.ds(g * _L, _L)] = sv * cv

                def srow_body(r):
                    row = g * _L + r
                    lv = plsc.load_gather(
                        idx_v, [jnp.full((_L,), ch, jnp.int32),
                                jnp.full((_L,), g * _L, jnp.int32) + r])
                    sval = plsc.load_gather(
                        sbuf, [jnp.full((_L,), g * _L, jnp.int32) + r])
                    for j in range(d // _L):
                        plsc.addupdate_scatter(
                            acc, [lv, lane + (j * _L)],
                            xbuf[b, row, pl.ds(j * _L, _L)] * sval)

                plsc.parallel_loop(0, _L, unroll=2)(srow_body)
            pltpu.sync_copy(scbuf,
                            sc_hbm.at[pl.ds(base + ch * chunk, chunk)])
        pltpu.sync_copy(acc, vec_hbm.at[wid])

    def _update_body(vec_ref, sc_ref, labr_ref, counts_ref, mu_ref,
                     text_ref, new_ref, loss_ref):
        ii = lax.broadcasted_iota(jnp.int32, (C, n), 0)
        mask = ii == labr_ref[...]
        scal = jnp.sum(jnp.where(mask, sc_ref[...], 0.0), axis=1,
                       keepdims=True)
        v = jnp.sum(vec_ref[...], axis=0)
        cnt = jnp.maximum(counts_ref[...], 1.0)
        mu = mu_ref[...]
        tang = v / cnt - (scal / cnt) * mu
        vn = jnp.sqrt(jnp.sum(tang * tang, axis=1, keepdims=True))
        vn = jnp.maximum(vn, _EPS)
        nm = jnp.cos(vn) * mu + jnp.sin(vn) * tang / vn
        nrm = jnp.sqrt(jnp.sum(nm * nm, axis=1, keepdims=True))
        newmu = nm / jnp.maximum(nrm, 1e-12)
        new_ref[...] = newmu
        dt = jnp.sum(newmu * text_ref[...], axis=1, keepdims=True)
        geo = _acos(jnp.clip(dt, -1.0 + _EPS, 1.0 - _EPS)) ** 2
        pres = (counts_ref[...] > 0.0).astype(jnp.float32)
        loss_ref[...] = (-jnp.sum(geo * pres) /
                         jnp.sum(pres)).reshape(1, 1)

    update_call = pl.pallas_call(
        _update_body,
        out_shape=[jax.ShapeDtypeStruct((C, d), jnp.float32),
                   jax.ShapeDtypeStruct((1, 1), jnp.float32)],
    )

    def pipeline(logits, img_feats, text_feats):
        labels2, counts = labels_call(logits)
        labels = labels2.reshape(n)
        labels_r = labels2.reshape(1, n)
        counts_c1 = counts.reshape(C, 1)
        sums = classsum_call(img_feats, labels)
        mu = mu0_call(sums, counts_c1)
        for _ in range(_N_ITER):
            vecsum, sc_arr = karcher_call(img_feats, labels, mu)
            mu, loss = update_call(vecsum, sc_arr.reshape(1, n), labels_r,
                                   counts_c1, mu, text_feats)
        return loss.reshape(())

    return pipeline


def kernel(logits, img_feats, text_feats):
    n, C = logits.shape
    d = img_feats.shape[1]
    return _make_pipeline(n, C, d)(logits, img_feats, text_feats)
